# Initial kernel scaffold; baseline (speedup 1.0000x reference)
import jax
import jax.numpy as jnp
from jax.experimental import pallas as pl


def kernel(x_nchw, weight, bias, gamma, beta):
    raise NotImplementedError("write your pallas kernel here")


# speedup vs baseline: 2.1974x; 2.1974x over previous
import functools

import jax
import jax.numpy as jnp
from jax import lax
from jax.experimental import pallas as pl
from jax.experimental.pallas import tpu as pltpu

_BN_EPS = 1e-5


def _make_patches(x, hw, w_sz):
    cin = x.shape[0]
    col = lax.broadcasted_iota(jnp.int32, (cin, hw), 1) % w_sz
    zero = jnp.zeros((), x.dtype)
    taps = []
    for di in range(3):
        for dj in range(3):
            o = (di - 1) * w_sz + (dj - 1)
            if o > 0:
                s = jnp.concatenate(
                    [x[:, o:], jnp.zeros((cin, o), x.dtype)], axis=1)
            elif o < 0:
                s = jnp.concatenate(
                    [jnp.zeros((cin, -o), x.dtype), x[:, :hw + o]], axis=1)
            else:
                s = x
            if dj == 0:
                s = jnp.where(col >= 1, s, zero)
            elif dj == 2:
                s = jnp.where(col <= w_sz - 2, s, zero)
            taps.append(s)
    return jnp.concatenate(taps, axis=0)


def _conv_stats_kernel(x_ref, w_ref, stats_ref, *, hw, w_sz):
    patches = _make_patches(x_ref[0], hw, w_sz)
    y = jnp.dot(w_ref[...], patches, preferred_element_type=jnp.float32)
    s1 = jnp.sum(y, axis=1, keepdims=True)
    s2 = jnp.sum(y * y, axis=1, keepdims=True)
    st = jnp.concatenate([s1, s2], axis=1)
    stats_ref[0] = jnp.transpose(st)


def _conv_bn_relu_kernel(x_ref, w_ref, ps_ref, g_ref, b_ref, o_ref, *,
                         hw, w_sz, inv_count):
    patches = _make_patches(x_ref[0], hw, w_sz)
    y = jnp.dot(w_ref[...], patches, preferred_element_type=jnp.float32)

    tot = jnp.sum(ps_ref[...], axis=0)
    mean = tot[0:1, :] * inv_count
    var = jnp.maximum(tot[1:2, :] * inv_count - mean * mean, 0.0)
    inv = lax.rsqrt(var + _BN_EPS)
    scale = jnp.transpose(g_ref[...] * inv)
    shift = jnp.transpose(b_ref[...] - mean * (g_ref[...] * inv))
    o_ref[0] = jnp.maximum(y * scale + shift, 0.0)


def kernel(x_nchw, weight, bias, gamma, beta):
    del bias
    n, cin, h, w = x_nchw.shape
    cout = weight.shape[0]
    hw = h * w
    m = n * hw

    x = x_nchw.reshape(n, cin, hw).astype(jnp.float32)
    w_mat = jnp.transpose(weight, (0, 2, 3, 1)).reshape(cout, 9 * cin)
    w_mat = w_mat.astype(jnp.float32)
    g_row = gamma.reshape(1, cout).astype(jnp.float32)
    b_row = beta.reshape(1, cout).astype(jnp.float32)

    k1 = functools.partial(_conv_stats_kernel, hw=hw, w_sz=w)
    partial_stats = pl.pallas_call(
        k1,
        out_shape=jax.ShapeDtypeStruct((n, 2, cout), jnp.float32),
        grid=(n,),
        in_specs=[
            pl.BlockSpec((1, cin, hw), lambda i: (i, 0, 0)),
            pl.BlockSpec((cout, 9 * cin), lambda i: (0, 0)),
        ],
        out_specs=pl.BlockSpec((1, 2, cout), lambda i: (i, 0, 0)),
        compiler_params=pltpu.CompilerParams(
            dimension_semantics=("parallel",)),
    )(x, w_mat)

    k2 = functools.partial(_conv_bn_relu_kernel, hw=hw, w_sz=w,
                           inv_count=1.0 / m)
    out = pl.pallas_call(
        k2,
        out_shape=jax.ShapeDtypeStruct((n, cout, hw), jnp.float32),
        grid=(n,),
        in_specs=[
            pl.BlockSpec((1, cin, hw), lambda i: (i, 0, 0)),
            pl.BlockSpec((cout, 9 * cin), lambda i: (0, 0)),
            pl.BlockSpec((n, 2, cout), lambda i: (0, 0, 0)),
            pl.BlockSpec((1, cout), lambda i: (0, 0)),
            pl.BlockSpec((1, cout), lambda i: (0, 0)),
        ],
        out_specs=pl.BlockSpec((1, cout, hw), lambda i: (i, 0, 0)),
        compiler_params=pltpu.CompilerParams(
            dimension_semantics=("parallel",)),
    )(x, w_mat, partial_stats, g_row, b_row)

    return out.reshape(n, cout, h, w)


# speedup vs baseline: 2.8673x; 1.3048x over previous
import functools

import jax
import jax.numpy as jnp
from jax import lax
from jax.experimental import pallas as pl
from jax.experimental.pallas import tpu as pltpu

_BN_EPS = 1e-5


def _make_patches(x, hw, w_sz):
    cin = x.shape[0]
    col = lax.broadcasted_iota(jnp.int32, (cin, hw), 1) % w_sz
    zero = jnp.zeros((), x.dtype)
    taps = []
    for di in range(3):
        for dj in range(3):
            o = (di - 1) * w_sz + (dj - 1)
            if o > 0:
                s = jnp.concatenate(
                    [x[:, o:], jnp.zeros((cin, o), x.dtype)], axis=1)
            elif o < 0:
                s = jnp.concatenate(
                    [jnp.zeros((cin, -o), x.dtype), x[:, :hw + o]], axis=1)
            else:
                s = x
            if dj == 0:
                s = jnp.where(col >= 1, s, zero)
            elif dj == 2:
                s = jnp.where(col <= w_sz - 2, s, zero)
            taps.append(s)
    return jnp.concatenate(taps, axis=0)


def _conv_stats_kernel(x_ref, w_ref, stats_ref, *, hw, w_sz, nb):
    s1 = jnp.zeros((w_ref.shape[0], 1), jnp.float32)
    s2 = s1
    for b in range(nb):
        patches = _make_patches(x_ref[b], hw, w_sz)
        y = jnp.dot(w_ref[...], patches, preferred_element_type=jnp.float32)
        s1 = s1 + jnp.sum(y, axis=1, keepdims=True)
        s2 = s2 + jnp.sum(y * y, axis=1, keepdims=True)
    st = jnp.concatenate([s1, s2], axis=1)
    stats_ref[0] = jnp.transpose(st)


def _conv_bn_relu_kernel(x_ref, w_ref, ps_ref, g_ref, b_ref, o_ref, *,
                         hw, w_sz, inv_count, nb):
    tot = jnp.sum(ps_ref[...], axis=0)
    mean = tot[0:1, :] * inv_count
    var = jnp.maximum(tot[1:2, :] * inv_count - mean * mean, 0.0)
    inv = lax.rsqrt(var + _BN_EPS)
    sc = g_ref[...] * inv
    scale = jnp.transpose(sc)
    shift = jnp.transpose(b_ref[...] - mean * sc)
    for b in range(nb):
        patches = _make_patches(x_ref[b], hw, w_sz)
        y = jnp.dot(w_ref[...], patches, preferred_element_type=jnp.float32)
        o_ref[b] = jnp.maximum(y * scale + shift, 0.0)


def kernel(x_nchw, weight, bias, gamma, beta):
    del bias
    n, cin, h, w = x_nchw.shape
    cout = weight.shape[0]
    hw = h * w
    m = n * hw

    nb = 4 if n % 4 == 0 else 1
    ng = n // nb

    x = x_nchw.reshape(n, cin, hw).astype(jnp.bfloat16)
    w_mat = jnp.transpose(weight, (0, 2, 3, 1)).reshape(cout, 9 * cin)
    w_mat = w_mat.astype(jnp.bfloat16)
    g_row = gamma.reshape(1, cout).astype(jnp.float32)
    b_row = beta.reshape(1, cout).astype(jnp.float32)

    k1 = functools.partial(_conv_stats_kernel, hw=hw, w_sz=w, nb=nb)
    partial_stats = pl.pallas_call(
        k1,
        out_shape=jax.ShapeDtypeStruct((ng, 2, cout), jnp.float32),
        grid=(ng,),
        in_specs=[
            pl.BlockSpec((nb, cin, hw), lambda i: (i, 0, 0)),
            pl.BlockSpec((cout, 9 * cin), lambda i: (0, 0)),
        ],
        out_specs=pl.BlockSpec((1, 2, cout), lambda i: (i, 0, 0)),
        compiler_params=pltpu.CompilerParams(
            dimension_semantics=("parallel",)),
    )(x, w_mat)

    k2 = functools.partial(_conv_bn_relu_kernel, hw=hw, w_sz=w,
                           inv_count=1.0 / m, nb=nb)
    out = pl.pallas_call(
        k2,
        out_shape=jax.ShapeDtypeStruct((n, cout, hw), jnp.float32),
        grid=(ng,),
        in_specs=[
            pl.BlockSpec((nb, cin, hw), lambda i: (i, 0, 0)),
            pl.BlockSpec((cout, 9 * cin), lambda i: (0, 0)),
            pl.BlockSpec((ng, 2, cout), lambda i: (0, 0, 0)),
            pl.BlockSpec((1, cout), lambda i: (0, 0)),
            pl.BlockSpec((1, cout), lambda i: (0, 0)),
        ],
        out_specs=pl.BlockSpec((nb, cout, hw), lambda i: (i, 0, 0)),
        compiler_params=pltpu.CompilerParams(
            dimension_semantics=("parallel",)),
    )(x, w_mat, partial_stats, g_row, b_row)

    return out.reshape(n, cout, h, w)


# speedup vs baseline: 3.5182x; 1.2270x over previous
import functools

import jax
import jax.numpy as jnp
from jax import lax
from jax.experimental import pallas as pl
from jax.experimental.pallas import tpu as pltpu

_BN_EPS = 1e-5


def _make_patches(x, hw, w_sz):
    cin = x.shape[0]
    col = lax.broadcasted_iota(jnp.int32, (cin, hw), 1) % w_sz
    zero = jnp.zeros((), x.dtype)
    taps = []
    for di in range(3):
        for dj in range(3):
            o = (di - 1) * w_sz + (dj - 1)
            if o > 0:
                s = jnp.concatenate(
                    [x[:, o:], jnp.zeros((cin, o), x.dtype)], axis=1)
            elif o < 0:
                s = jnp.concatenate(
                    [jnp.zeros((cin, -o), x.dtype), x[:, :hw + o]], axis=1)
            else:
                s = x
            if dj == 0:
                s = jnp.where(col >= 1, s, zero)
            elif dj == 2:
                s = jnp.where(col <= w_sz - 2, s, zero)
            taps.append(s)
    return jnp.concatenate(taps, axis=0)


def _conv_stats_kernel(x_ref, w_ref, stats_ref, *, hw, w_sz, nb):
    s1 = jnp.zeros((w_ref.shape[0], 1), jnp.float32)
    s2 = s1
    for b in range(nb):
        patches = _make_patches(x_ref[b], hw, w_sz)
        y = jnp.dot(w_ref[...], patches, preferred_element_type=jnp.float32)
        s1 = s1 + jnp.sum(y, axis=1, keepdims=True)
        s2 = s2 + jnp.sum(y * y, axis=1, keepdims=True)
    st = jnp.concatenate([s1, s2], axis=1)
    stats_ref[0] = jnp.transpose(st)


def _conv_bn_relu_kernel(x_ref, w_ref, ps_ref, g_ref, b_ref, o_ref, *,
                         hw, w_sz, inv_count, nb):
    tot = jnp.sum(ps_ref[...], axis=0)
    mean = tot[0:1, :] * inv_count
    var = jnp.maximum(tot[1:2, :] * inv_count - mean * mean, 0.0)
    inv = lax.rsqrt(var + _BN_EPS)
    sc = g_ref[...] * inv
    scale = jnp.transpose(sc)
    shift = jnp.transpose(b_ref[...] - mean * sc)
    for b in range(nb):
        patches = _make_patches(x_ref[b], hw, w_sz)
        y = jnp.dot(w_ref[...], patches, preferred_element_type=jnp.float32)
        y = jnp.maximum(y * scale + shift, 0.0)
        o_ref[b] = jnp.transpose(y)


def kernel(x_nchw, weight, bias, gamma, beta):
    del bias
    n, cin, h, w = x_nchw.shape
    cout = weight.shape[0]
    hw = h * w
    m = n * hw

    nb = 4 if n % 4 == 0 else 1
    ng = n // nb

    x = x_nchw.reshape(n, cin, hw).astype(jnp.bfloat16)
    w_mat = jnp.transpose(weight, (0, 2, 3, 1)).reshape(cout, 9 * cin)
    w_mat = w_mat.astype(jnp.bfloat16)
    g_row = gamma.reshape(1, cout).astype(jnp.float32)
    b_row = beta.reshape(1, cout).astype(jnp.float32)

    k1 = functools.partial(_conv_stats_kernel, hw=hw, w_sz=w, nb=nb)
    partial_stats = pl.pallas_call(
        k1,
        out_shape=jax.ShapeDtypeStruct((ng, 2, cout), jnp.float32),
        grid=(ng,),
        in_specs=[
            pl.BlockSpec((nb, cin, hw), lambda i: (i, 0, 0)),
            pl.BlockSpec((cout, 9 * cin), lambda i: (0, 0)),
        ],
        out_specs=pl.BlockSpec((1, 2, cout), lambda i: (i, 0, 0)),
        compiler_params=pltpu.CompilerParams(
            dimension_semantics=("parallel",)),
    )(x, w_mat)

    k2 = functools.partial(_conv_bn_relu_kernel, hw=hw, w_sz=w,
                           inv_count=1.0 / m, nb=nb)
    out = pl.pallas_call(
        k2,
        out_shape=jax.ShapeDtypeStruct((n, hw, cout), jnp.float32),
        grid=(ng,),
        in_specs=[
            pl.BlockSpec((nb, cin, hw), lambda i: (i, 0, 0)),
            pl.BlockSpec((cout, 9 * cin), lambda i: (0, 0)),
            pl.BlockSpec((ng, 2, cout), lambda i: (0, 0, 0)),
            pl.BlockSpec((1, cout), lambda i: (0, 0)),
            pl.BlockSpec((1, cout), lambda i: (0, 0)),
        ],
        out_specs=pl.BlockSpec((nb, hw, cout), lambda i: (i, 0, 0)),
        compiler_params=pltpu.CompilerParams(
            dimension_semantics=("parallel",)),
    )(x, w_mat, partial_stats, g_row, b_row)

    return jnp.transpose(out.reshape(n, h, w, cout), (0, 3, 1, 2))
